# baseline (speedup 1.0000x reference)
import jax
import jax.numpy as jnp
from jax import lax
from jax.experimental import pallas as pl
from jax.experimental.pallas import tpu as pltpu
from jax.experimental.pallas import tpu_sc as plsc

B = 4
S = 4096
D = 1024
N = B * S

NUM_CORES = 2
NUM_SUBCORES = 16
NW = NUM_CORES * NUM_SUBCORES
POS_PER_W = S // NW
CP = 8
NCHUNK = POS_PER_W // CP
RPC = B * CP
LANES = 16
VPR = D // LANES
NIDX = B * POS_PER_W


def _body(ids_hbm, tok_hbm, pos_hbm, out_hbm,
          idx_v, tok0, tok1, pos0, pos1,
          isem, lsem0, lsem1, gsem0, gsem1, wsem0, wsem1):
    wid = lax.axis_index("s") * NUM_CORES + lax.axis_index("c")
    pbase = wid * POS_PER_W

    toks = [tok0, tok1]
    poss = [pos0, pos1]
    lsems = [lsem0, lsem1]
    gsems = [gsem0, gsem1]
    wsems = [wsem0, wsem1]

    descs = []
    for b in range(B):
        descs.append(pltpu.async_copy(
            ids_hbm.at[pl.ds(b * S + pbase, POS_PER_W)],
            idx_v.at[pl.ds(b * POS_PER_W, POS_PER_W)], isem))
    for d in descs:
        d.wait()

    def fire_loads(c, k):
        pltpu.async_copy(
            pos_hbm.at[pl.ds(pbase + c * CP, CP)], poss[k], lsems[k])
        for b in range(B):
            pltpu.async_copy(
                tok_hbm.at[idx_v.at[pl.ds(b * POS_PER_W + c * CP, CP)]],
                toks[k].at[pl.ds(b * CP, CP)], gsems[k])

    def wait_loads(c, k):
        pltpu.make_async_copy(
            pos_hbm.at[pl.ds(pbase + c * CP, CP)], poss[k], lsems[k]).wait()
        for b in range(B):
            pltpu.make_async_copy(
                tok_hbm.at[idx_v.at[pl.ds(b * POS_PER_W + c * CP, CP)]],
                toks[k].at[pl.ds(b * CP, CP)], gsems[k]).wait()

    def fire_wb(c, k):
        for b in range(B):
            pltpu.async_copy(
                toks[k].at[pl.ds(b * CP, CP)],
                out_hbm.at[pl.ds(b * S + pbase + c * CP, CP)], wsems[k])

    def wait_wb(c, k):
        for b in range(B):
            pltpu.make_async_copy(
                toks[k].at[pl.ds(b * CP, CP)],
                out_hbm.at[pl.ds(b * S + pbase + c * CP, CP)], wsems[k]).wait()

    def process(c, k):
        @pl.when(c + 1 < NCHUNK)
        def _():
            @pl.when(c >= 1)
            def _():
                wait_wb(c - 1, k ^ 1)
            fire_loads(c + 1, k ^ 1)

        wait_loads(c, k)
        tok_v = toks[k]
        pos_v = poss[k]
        for b in range(B):
            def add_row(r, carry):
                for j in range(VPR):
                    x = pos_v[r, pl.ds(j * LANES, LANES)]
                    plsc.addupdate(
                        tok_v.at[b * CP + r, pl.ds(j * LANES, LANES)], x)
                return carry

            lax.fori_loop(0, CP, add_row, 0)
            pltpu.async_copy(
                tok_v.at[pl.ds(b * CP, CP)],
                out_hbm.at[pl.ds(b * S + pbase + c * CP, CP)], wsems[k])

    fire_loads(0, 0)

    @pl.loop(0, NCHUNK // 2)
    def _chunks(g):
        process(2 * g, 0)
        process(2 * g + 1, 1)

    wait_wb(NCHUNK - 2, 0)
    wait_wb(NCHUNK - 1, 1)


@jax.jit
def _run(ids_flat, token_emb, pos_emb):
    mesh = plsc.VectorSubcoreMesh(
        core_axis_name="c", subcore_axis_name="s",
        num_cores=NUM_CORES, num_subcores=NUM_SUBCORES,
    )
    return pl.kernel(
        _body,
        out_type=jax.ShapeDtypeStruct((N, D), jnp.float32),
        mesh=mesh,
        scratch_types=[
            pltpu.VMEM((NIDX,), jnp.int32),
            pltpu.VMEM((RPC, D), jnp.float32),
            pltpu.VMEM((RPC, D), jnp.float32),
            pltpu.VMEM((CP, D), jnp.float32),
            pltpu.VMEM((CP, D), jnp.float32),
            pltpu.SemaphoreType.DMA,
            pltpu.SemaphoreType.DMA,
            pltpu.SemaphoreType.DMA,
            pltpu.SemaphoreType.DMA,
            pltpu.SemaphoreType.DMA,
            pltpu.SemaphoreType.DMA,
            pltpu.SemaphoreType.DMA,
        ],
    )(ids_flat, token_emb, pos_emb)


def kernel(token_ids, token_emb, pos_emb):
    ids_flat = token_ids.reshape(-1).astype(jnp.int32)
    out = _run(ids_flat, token_emb, pos_emb)
    return out.reshape(B, S, D)


# speedup vs baseline: 1.0467x; 1.0467x over previous
import jax
import jax.numpy as jnp
from jax import lax
from jax.experimental import pallas as pl
from jax.experimental.pallas import tpu as pltpu
from jax.experimental.pallas import tpu_sc as plsc

B = 4
S = 4096
D = 1024
N = B * S

NUM_CORES = 2
NUM_SUBCORES = 16
NW = NUM_CORES * NUM_SUBCORES
POS_PER_W = S // NW
CP = 8
NCHUNK = POS_PER_W // CP
RPC = B * CP
LANES = 16
VPR = D // LANES
NIDX = B * POS_PER_W


def _body(ids_hbm, tok_hbm, pos_hbm, out_hbm,
          idx_v, tok0, tok1, pos0, pos1,
          isem, lsem0, lsem1, gsem0, gsem1, wsem0, wsem1):
    wid = lax.axis_index("s") * NUM_CORES + lax.axis_index("c")
    pbase = wid * POS_PER_W

    toks = [tok0, tok1]
    poss = [pos0, pos1]
    lsems = [lsem0, lsem1]
    gsems = [gsem0, gsem1]
    wsems = [wsem0, wsem1]

    descs = []
    for b in range(B):
        descs.append(pltpu.async_copy(
            ids_hbm.at[pl.ds(b * S + pbase, POS_PER_W)],
            idx_v.at[pl.ds(b * POS_PER_W, POS_PER_W)], isem))
    for d in descs:
        d.wait()

    def fire_loads(c, k):
        pltpu.async_copy(
            pos_hbm.at[pl.ds(pbase + c * CP, CP)], poss[k], lsems[k])
        for b in range(B):
            pltpu.async_copy(
                tok_hbm.at[idx_v.at[pl.ds(b * POS_PER_W + c * CP, CP)]],
                toks[k].at[pl.ds(b * CP, CP)], gsems[k])

    def wait_loads(c, k):
        pltpu.make_async_copy(
            pos_hbm.at[pl.ds(pbase + c * CP, CP)], poss[k], lsems[k]).wait()
        for b in range(B):
            pltpu.make_async_copy(
                tok_hbm.at[idx_v.at[pl.ds(b * POS_PER_W + c * CP, CP)]],
                toks[k].at[pl.ds(b * CP, CP)], gsems[k]).wait()

    def fire_wb(c, k):
        for b in range(B):
            pltpu.async_copy(
                toks[k].at[pl.ds(b * CP, CP)],
                out_hbm.at[pl.ds(b * S + pbase + c * CP, CP)], wsems[k])

    def wait_wb(c, k):
        for b in range(B):
            pltpu.make_async_copy(
                toks[k].at[pl.ds(b * CP, CP)],
                out_hbm.at[pl.ds(b * S + pbase + c * CP, CP)], wsems[k]).wait()

    def process(c, k):
        @pl.when(c + 1 < NCHUNK)
        def _():
            @pl.when(c >= 1)
            def _():
                wait_wb(c - 1, k ^ 1)
            fire_loads(c + 1, k ^ 1)

        wait_loads(c, k)
        tok_v = toks[k]
        pos_v = poss[k]

        def add_row(r, carry):
            for j in range(VPR):
                x = pos_v[r, pl.ds(j * LANES, LANES)]
                for b in range(B):
                    plsc.addupdate(
                        tok_v.at[b * CP + r, pl.ds(j * LANES, LANES)], x)
            return carry

        lax.fori_loop(0, CP, add_row, 0)
        fire_wb(c, k)

    fire_loads(0, 0)

    @pl.loop(0, NCHUNK // 2)
    def _chunks(g):
        process(2 * g, 0)
        process(2 * g + 1, 1)

    wait_wb(NCHUNK - 2, 0)
    wait_wb(NCHUNK - 1, 1)


@jax.jit
def _run(ids_flat, token_emb, pos_emb):
    mesh = plsc.VectorSubcoreMesh(
        core_axis_name="c", subcore_axis_name="s",
        num_cores=NUM_CORES, num_subcores=NUM_SUBCORES,
    )
    return pl.kernel(
        _body,
        out_type=jax.ShapeDtypeStruct((N, D), jnp.float32),
        mesh=mesh,
        scratch_types=[
            pltpu.VMEM((NIDX,), jnp.int32),
            pltpu.VMEM((RPC, D), jnp.float32),
            pltpu.VMEM((RPC, D), jnp.float32),
            pltpu.VMEM((CP, D), jnp.float32),
            pltpu.VMEM((CP, D), jnp.float32),
            pltpu.SemaphoreType.DMA,
            pltpu.SemaphoreType.DMA,
            pltpu.SemaphoreType.DMA,
            pltpu.SemaphoreType.DMA,
            pltpu.SemaphoreType.DMA,
            pltpu.SemaphoreType.DMA,
            pltpu.SemaphoreType.DMA,
        ],
    )(ids_flat, token_emb, pos_emb)


def kernel(token_ids, token_emb, pos_emb):
    ids_flat = token_ids.reshape(-1).astype(jnp.int32)
    out = _run(ids_flat, token_emb, pos_emb)
    return out.reshape(B, S, D)
